# baseline (speedup 1.0000x reference)
import functools

import jax
import jax.numpy as jnp
from jax import lax
from jax.experimental import pallas as pl
from jax.experimental.pallas import tpu as pltpu
from jax.experimental.pallas import tpu_sc as plsc

_N_EXPERTS = 64
_TOP_K = 8
_LANE = 128
_SC_TOK = 1024


def _logits_block(x_ref, w_ref, b_ref, lg_ref, z_ref):
    bias_col = jnp.transpose(b_ref[...], (1, 0))
    logits = jax.lax.dot_general(
        w_ref[...], x_ref[0],
        (((1,), (1,)), ((), ())),
        preferred_element_type=jnp.float32,
    ) + bias_col

    t = logits.shape[1]
    lg_ref[...] = jnp.transpose(
        logits.reshape(_N_EXPERTS, t // _LANE, _LANE), (1, 0, 2))

    m0 = jnp.max(logits, axis=0, keepdims=True)
    se = jnp.sum(jnp.exp(logits - m0), axis=0, keepdims=True)
    lse = m0 + jnp.log(se)

    @pl.when((pl.program_id(0) == 0) & (pl.program_id(1) == 0))
    def _():
        z_ref[...] = jnp.zeros((1, 1), jnp.float32)

    z_ref[...] += jnp.sum(lse * lse, axis=1, keepdims=True)


def _sc_route(lg_hbm, prob_hbm, idx_hbm, lg_v, pv, iv, sem):
    c = lax.axis_index("c")
    sidx = lax.axis_index("s")
    wid = sidx * 2 + c
    g0 = wid * (_SC_TOK // _LANE)

    pltpu.sync_copy(lg_hbm.at[pl.ds(g0, _SC_TOK // _LANE)], lg_v)

    def body(it, carry):
        g = it // (_LANE // 16)
        c0 = (it % (_LANE // 16)) * 16

        vals = [jnp.full((16,), -jnp.inf, jnp.float32) for _ in range(_TOP_K)]
        idxs = [jnp.zeros((16,), jnp.int32) for _ in range(_TOP_K)]
        for e in range(_N_EXPERTS):
            nv = lg_v[g, e, pl.ds(c0, 16)]
            ni = jnp.full((16,), e, jnp.int32)
            for j in range(_TOP_K):
                m = nv > vals[j]
                vals[j], nv = (jnp.where(m, nv, vals[j]),
                               jnp.where(m, vals[j], nv))
                idxs[j], ni = (jnp.where(m, ni, idxs[j]),
                               jnp.where(m, idxs[j], ni))

        es = [jnp.exp(v - vals[0]) for v in vals]
        tot = es[0]
        for v in es[1:]:
            tot = tot + v
        rcp = jnp.float32(1.0) / tot
        off = g * _LANE + c0
        for j in range(_TOP_K):
            pv[j, pl.ds(off, 16)] = es[j] * rcp
            iv[j, pl.ds(off, 16)] = idxs[j]
        return carry

    lax.fori_loop(0, (_SC_TOK // _LANE) * (_LANE // 16), body, jnp.int32(0))

    b = wid // 8
    s0 = (wid % 8) * _SC_TOK
    for j in range(_TOP_K):
        pltpu.sync_copy(pv.at[j], prob_hbm.at[b, j, pl.ds(s0, _SC_TOK)])
        pltpu.sync_copy(iv.at[j], idx_hbm.at[b, j, pl.ds(s0, _SC_TOK)])


@functools.partial(jax.jit, static_argnames=())
def kernel(x, W, expert_bias):
    b, s, d = x.shape
    block_t = 4096
    grid = (b, s // block_t)
    n_groups = b * s // _LANE

    lg, zsum = pl.pallas_call(
        _logits_block,
        grid=grid,
        in_specs=[
            pl.BlockSpec((1, block_t, d), lambda i, j: (i, j, 0)),
            pl.BlockSpec((_N_EXPERTS, d), lambda i, j: (0, 0)),
            pl.BlockSpec((1, _N_EXPERTS), lambda i, j: (0, 0)),
        ],
        out_specs=[
            pl.BlockSpec((block_t // _LANE, _N_EXPERTS, _LANE),
                         lambda i, j: (2 * i + j, 0, 0)),
            pl.BlockSpec((1, 1), lambda i, j: (0, 0)),
        ],
        out_shape=[
            jax.ShapeDtypeStruct((n_groups, _N_EXPERTS, _LANE), jnp.float32),
            jax.ShapeDtypeStruct((1, 1), jnp.float32),
        ],
        compiler_params=pltpu.CompilerParams(
            dimension_semantics=("arbitrary", "arbitrary"),
        ),
    )(x, W, expert_bias.reshape(1, _N_EXPERTS))

    mesh = plsc.VectorSubcoreMesh(core_axis_name="c", subcore_axis_name="s")
    sc = functools.partial(
        pl.kernel, mesh=mesh,
        out_type=[
            jax.ShapeDtypeStruct((b, _TOP_K, s), jnp.float32),
            jax.ShapeDtypeStruct((b, _TOP_K, s), jnp.int32),
        ],
        scratch_types=[
            pltpu.VMEM((_SC_TOK // _LANE, _N_EXPERTS, _LANE), jnp.float32),
            pltpu.VMEM((_TOP_K, _SC_TOK), jnp.float32),
            pltpu.VMEM((_TOP_K, _SC_TOK), jnp.int32),
            pltpu.SemaphoreType.DMA,
        ],
    )(_sc_route)

    prob, idx = sc(lg)

    return (prob.transpose(0, 2, 1),
            idx.transpose(0, 2, 1),
            zsum[0, 0] / jnp.float32(b * s))


# speedup vs baseline: 2.0909x; 2.0909x over previous
import functools

import jax
import jax.numpy as jnp
from jax.experimental import pallas as pl
from jax.experimental.pallas import tpu as pltpu

_N_EXPERTS = 64
_TOP_K = 8


def _router_block(x_ref, w_ref, b_ref, prob_ref, idx_ref, z_ref):
    bias_col = jnp.transpose(b_ref[...], (1, 0))
    logits = jax.lax.dot_general(
        w_ref[...], x_ref[0],
        (((1,), (1,)), ((), ())),
        preferred_element_type=jnp.float32,
    ) + bias_col

    t = logits.shape[1]
    e_iota = jax.lax.broadcasted_iota(jnp.int32, (_N_EXPERTS, t), 0)
    neg_inf = jnp.float32(-jnp.inf)

    vals = logits
    maxes, argmaxes = [], []
    for _ in range(_TOP_K):
        m = jnp.max(vals, axis=0, keepdims=True)
        hit = vals == m
        a = jnp.min(jnp.where(hit, e_iota, jnp.int32(_N_EXPERTS)),
                    axis=0, keepdims=True)
        maxes.append(m)
        argmaxes.append(a)
        vals = jnp.where(hit & (e_iota == a), neg_inf, vals)
    scores = jnp.concatenate(maxes, axis=0)
    idx = jnp.concatenate(argmaxes, axis=0)

    m0 = scores[0:1, :]
    p = jnp.exp(scores - m0)
    prob_ref[0] = p / jnp.sum(p, axis=0, keepdims=True)
    idx_ref[0] = idx

    se = jnp.sum(jnp.exp(logits - m0), axis=0, keepdims=True)
    lse = m0 + jnp.log(se)

    @pl.when((pl.program_id(0) == 0) & (pl.program_id(1) == 0))
    def _():
        z_ref[...] = jnp.zeros((1, 1), jnp.float32)

    z_ref[...] += jnp.sum(lse * lse, axis=1, keepdims=True)


@functools.partial(jax.jit, static_argnames=())
def kernel(x, W, expert_bias):
    b, s, d = x.shape
    block_t = 4096
    grid = (b, s // block_t)

    prob, idx, zsum = pl.pallas_call(
        _router_block,
        grid=grid,
        in_specs=[
            pl.BlockSpec((1, block_t, d), lambda i, j: (i, j, 0)),
            pl.BlockSpec((_N_EXPERTS, d), lambda i, j: (0, 0)),
            pl.BlockSpec((1, _N_EXPERTS), lambda i, j: (0, 0)),
        ],
        out_specs=[
            pl.BlockSpec((1, _TOP_K, block_t), lambda i, j: (i, 0, j)),
            pl.BlockSpec((1, _TOP_K, block_t), lambda i, j: (i, 0, j)),
            pl.BlockSpec((1, 1), lambda i, j: (0, 0)),
        ],
        out_shape=[
            jax.ShapeDtypeStruct((b, _TOP_K, s), jnp.float32),
            jax.ShapeDtypeStruct((b, _TOP_K, s), jnp.int32),
            jax.ShapeDtypeStruct((1, 1), jnp.float32),
        ],
        compiler_params=pltpu.CompilerParams(
            dimension_semantics=("arbitrary", "arbitrary"),
        ),
    )(x, W, expert_bias.reshape(1, _N_EXPERTS))

    return (prob.transpose(0, 2, 1),
            idx.transpose(0, 2, 1),
            zsum[0, 0] / jnp.float32(b * s))


# speedup vs baseline: 2.2502x; 1.0762x over previous
import functools

import jax
import jax.numpy as jnp
from jax.experimental import pallas as pl
from jax.experimental.pallas import tpu as pltpu

_N_EXPERTS = 64
_TOP_K = 8


def _router_block(x_ref, w_ref, b_ref, prob_ref, idx_ref, z_ref):
    _INT_MIN = jnp.int32(-2147483648)

    bias_col = jnp.transpose(b_ref[...], (1, 0))
    logits = jax.lax.dot_general(
        w_ref[...], x_ref[0],
        (((1,), (1,)), ((), ())),
        preferred_element_type=jnp.float32,
    ) + bias_col

    t = logits.shape[1]

    bits = jax.lax.bitcast_convert_type(logits, jnp.int32)
    key = jnp.where(bits >= 0, bits, _INT_MIN - bits)
    e_iota = jax.lax.broadcasted_iota(jnp.int32, (_N_EXPERTS, t), 0)
    key = (key & jnp.int32(-64)) | (jnp.int32(63) - e_iota)

    maxes = []
    for _ in range(_TOP_K):
        m = jnp.max(key, axis=0, keepdims=True)
        maxes.append(m)
        key = jnp.where(key == m, _INT_MIN, key)
    kstack = jnp.concatenate(maxes, axis=0)

    idx = jnp.int32(63) - (kstack & jnp.int32(63))
    sbits = jnp.where(kstack >= 0, kstack, _INT_MIN - kstack)
    scores = jax.lax.bitcast_convert_type(sbits, jnp.float32)

    m0 = scores[0:1, :]
    p = jnp.exp(scores - m0)
    prob_ref[0] = p / jnp.sum(p, axis=0, keepdims=True)
    idx_ref[0] = idx

    se = jnp.sum(jnp.exp(logits - m0), axis=0, keepdims=True)
    lse = m0 + jnp.log(se)

    @pl.when((pl.program_id(0) == 0) & (pl.program_id(1) == 0))
    def _():
        z_ref[...] = jnp.zeros((1, 1), jnp.float32)

    z_ref[...] += jnp.sum(lse * lse, axis=1, keepdims=True)


@functools.partial(jax.jit, static_argnames=())
def kernel(x, W, expert_bias):
    b, s, d = x.shape
    block_t = 4096
    grid = (b, s // block_t)

    prob, idx, zsum = pl.pallas_call(
        _router_block,
        grid=grid,
        in_specs=[
            pl.BlockSpec((1, block_t, d), lambda i, j: (i, j, 0)),
            pl.BlockSpec((_N_EXPERTS, d), lambda i, j: (0, 0)),
            pl.BlockSpec((1, _N_EXPERTS), lambda i, j: (0, 0)),
        ],
        out_specs=[
            pl.BlockSpec((1, _TOP_K, block_t), lambda i, j: (i, 0, j)),
            pl.BlockSpec((1, _TOP_K, block_t), lambda i, j: (i, 0, j)),
            pl.BlockSpec((1, 1), lambda i, j: (0, 0)),
        ],
        out_shape=[
            jax.ShapeDtypeStruct((b, _TOP_K, s), jnp.float32),
            jax.ShapeDtypeStruct((b, _TOP_K, s), jnp.int32),
            jax.ShapeDtypeStruct((1, 1), jnp.float32),
        ],
        compiler_params=pltpu.CompilerParams(
            dimension_semantics=("arbitrary", "arbitrary"),
        ),
    )(x, W, expert_bias.reshape(1, _N_EXPERTS))

    return (prob.transpose(0, 2, 1),
            idx.transpose(0, 2, 1),
            zsum[0, 0] / jnp.float32(b * s))
